# baseline (speedup 1.0000x reference)
import functools

import jax
import jax.numpy as jnp
from jax import lax
from jax.experimental import pallas as pl
from jax.experimental.pallas import tpu as pltpu
from jax.experimental.pallas import tpu_sc as plsc

NUM_GROUPS = 32
MIN_COUNT = 10
NBINS = 2 * NUM_GROUPS
LANES = 16
NSUB = 16


def _side_stats(s_lo, s_hi, c_lo, c_hi):
    one = jnp.float32(1.0)
    zero = jnp.float32(0.0)
    onev = jnp.ones((16,), jnp.float32)
    zerov = jnp.zeros((16,), jnp.float32)
    inc_lo = c_lo >= jnp.float32(MIN_COUNT)
    inc_hi = c_hi >= jnp.float32(MIN_COUNT)
    avg_lo = s_lo / jnp.maximum(c_lo, onev)
    avg_hi = s_hi / jnp.maximum(c_hi, onev)
    n = (jnp.sum(jnp.where(inc_lo, one, zero))
         + jnp.sum(jnp.where(inc_hi, one, zero)))
    nv = jnp.full((16,), n, jnp.float32)
    ssum = (jnp.sum(jnp.where(inc_lo, avg_lo, zerov))
            + jnp.sum(jnp.where(inc_hi, avg_hi, zerov)))
    mean_v = jnp.full((16,), ssum, jnp.float32) / jnp.maximum(nv, onev)
    d_lo = avg_lo - mean_v
    d_hi = avg_hi - mean_v
    dsum = (jnp.sum(jnp.where(inc_lo, d_lo * d_lo, zerov))
            + jnp.sum(jnp.where(inc_hi, d_hi * d_hi, zerov)))
    var_v = (jnp.full((16,), dsum, jnp.float32)
             / jnp.maximum(nv - onev, onev))
    return var_v, n >= jnp.float32(2.0)


def _make_sc_call(batch):
    chunk = batch // NSUB
    nvec = chunk // LANES
    mesh = plsc.VectorSubcoreMesh(core_axis_name="c", subcore_axis_name="s",
                                  num_cores=1)

    @functools.partial(
        pl.kernel,
        mesh=mesh,
        out_type=jax.ShapeDtypeStruct((LANES,), jnp.float32),
        compiler_params=pltpu.CompilerParams(needs_layout_passes=False),
        scratch_types=[
            pltpu.VMEM((chunk,), jnp.float32),
            pltpu.VMEM((chunk,), jnp.int32),
            pltpu.VMEM((chunk,), jnp.int32),
            pltpu.VMEM((LANES * NBINS,), jnp.float32),
            pltpu.VMEM((LANES * NBINS,), jnp.float32),
            pltpu.VMEM((NBINS,), jnp.float32),
            pltpu.VMEM((NBINS,), jnp.float32),
            pltpu.VMEM((LANES,), jnp.float32),
            pltpu.VMEM_SHARED((NSUB * NBINS,), jnp.float32),
            pltpu.VMEM_SHARED((NSUB * NBINS,), jnp.float32),
            pltpu.SemaphoreType.DMA,
            pltpu.SemaphoreType.DMA,
            pltpu.SemaphoreType.DMA,
        ],
    )
    def sc_loss(probs_hbm, labels_hbm, groups_hbm, out_hbm,
                pv, lv, gv, hsa, hca, ts, tc, ov, shs, shc,
                sem0, sem1, sem2):
        sid = lax.axis_index("s")
        lane = lax.iota(jnp.int32, LANES)
        zeros = jnp.zeros((LANES,), jnp.float32)
        ones = jnp.ones((LANES,), jnp.float32)
        nquart = NBINS // LANES

        base = sid * chunk
        half = chunk // 2

        def start_half(h, sem):
            o = base + h * half
            d = h * half
            c0 = pltpu.make_async_copy(
                probs_hbm.at[pl.ds(o, half)], pv.at[pl.ds(d, half)], sem)
            c1 = pltpu.make_async_copy(
                labels_hbm.at[pl.ds(o, half)], lv.at[pl.ds(d, half)], sem)
            c2 = pltpu.make_async_copy(
                groups_hbm.at[pl.ds(o, half)], gv.at[pl.ds(d, half)], sem)
            c0.start()
            c1.start()
            c2.start()
            return c0, c1, c2

        cps_a = start_half(0, sem0)
        cps_b = start_half(1, sem1)

        for v in range(LANES * nquart):
            hsa[pl.ds(v * LANES, LANES)] = zeros
            hca[pl.ds(v * LANES, LANES)] = zeros

        lane_off = lane * NBINS

        def step(i, carry):
            for u in range(4):
                off = (i * 4 + u) * LANES
                p = pv[pl.ds(off, LANES)]
                lb = lv[pl.ds(off, LANES)]
                g = gv[pl.ds(off, LANES)]
                slot = lane_off + g + lb * NUM_GROUPS
                plsc.addupdate_scatter(hsa, [slot], p)
                plsc.addupdate_scatter(hca, [slot], ones)
            return carry

        nvec_half = nvec // 2
        for c in cps_a:
            c.wait()
        lax.fori_loop(0, nvec_half // 4, step, 0)
        for c in cps_b:
            c.wait()
        lax.fori_loop(nvec_half // 4, nvec // 4, step, 0)

        for j in range(nquart):
            sacc = hsa[pl.ds(j * LANES, LANES)]
            cacc = hca[pl.ds(j * LANES, LANES)]
            for r in range(1, LANES):
                o = r * NBINS + j * LANES
                sacc = sacc + hsa[pl.ds(o, LANES)]
                cacc = cacc + hca[pl.ds(o, LANES)]
            ts[pl.ds(j * LANES, LANES)] = sacc
            tc[pl.ds(j * LANES, LANES)] = cacc

        cps = pltpu.make_async_copy(ts, shs.at[pl.ds(sid * NBINS, NBINS)], sem0)
        cpc = pltpu.make_async_copy(tc, shc.at[pl.ds(sid * NBINS, NBINS)], sem1)
        cps.start()
        cpc.start()
        cps.wait()
        cpc.wait()

        plsc.subcore_barrier()

        @pl.when(sid == 0)
        def _():
            cbs = pltpu.make_async_copy(shs, hsa, sem0)
            cbc = pltpu.make_async_copy(shc, hca, sem1)
            cbs.start()
            cbc.start()
            cbs.wait()
            cbc.wait()
            svecs = []
            cvecs = []
            for j in range(nquart):
                sacc = hsa[pl.ds(j * LANES, LANES)]
                cacc = hca[pl.ds(j * LANES, LANES)]
                for k in range(1, NSUB):
                    sacc = sacc + hsa[pl.ds(k * NBINS + j * LANES, LANES)]
                    cacc = cacc + hca[pl.ds(k * NBINS + j * LANES, LANES)]
                svecs.append(sacc)
                cvecs.append(cacc)
            neg_var, have_neg = _side_stats(svecs[0], svecs[1],
                                            cvecs[0], cvecs[1])
            pos_var, have_pos = _side_stats(svecs[2], svecs[3],
                                            cvecs[2], cvecs[3])
            zerov = jnp.zeros((LANES,), jnp.float32)
            loss_v = jnp.where(
                have_pos & have_neg,
                jnp.float32(0.5) * (pos_var + neg_var),
                jnp.where(have_pos, pos_var,
                          jnp.where(have_neg, neg_var, zerov)))
            ov[...] = loss_v
            pltpu.sync_copy(ov, out_hbm)

    return sc_loss


def kernel(probs, labels, groups, indices):
    del indices
    out = _make_sc_call(probs.shape[0])(probs, labels, groups)
    return out[0]


# speedup vs baseline: 1.0031x; 1.0031x over previous
import functools

import jax
import jax.numpy as jnp
from jax import lax
from jax.experimental import pallas as pl
from jax.experimental.pallas import tpu as pltpu
from jax.experimental.pallas import tpu_sc as plsc

NUM_GROUPS = 32
MIN_COUNT = 10
NBINS = 2 * NUM_GROUPS
LANES = 16
NSUB = 16


def _side_stats(s_lo, s_hi, c_lo, c_hi):
    one = jnp.float32(1.0)
    zero = jnp.float32(0.0)
    onev = jnp.ones((16,), jnp.float32)
    zerov = jnp.zeros((16,), jnp.float32)
    inc_lo = c_lo >= jnp.float32(MIN_COUNT)
    inc_hi = c_hi >= jnp.float32(MIN_COUNT)
    avg_lo = s_lo / jnp.maximum(c_lo, onev)
    avg_hi = s_hi / jnp.maximum(c_hi, onev)
    n = (jnp.sum(jnp.where(inc_lo, one, zero))
         + jnp.sum(jnp.where(inc_hi, one, zero)))
    nv = jnp.full((16,), n, jnp.float32)
    ssum = (jnp.sum(jnp.where(inc_lo, avg_lo, zerov))
            + jnp.sum(jnp.where(inc_hi, avg_hi, zerov)))
    mean_v = jnp.full((16,), ssum, jnp.float32) / jnp.maximum(nv, onev)
    d_lo = avg_lo - mean_v
    d_hi = avg_hi - mean_v
    dsum = (jnp.sum(jnp.where(inc_lo, d_lo * d_lo, zerov))
            + jnp.sum(jnp.where(inc_hi, d_hi * d_hi, zerov)))
    var_v = (jnp.full((16,), dsum, jnp.float32)
             / jnp.maximum(nv - onev, onev))
    return var_v, n >= jnp.float32(2.0)


def _make_sc_call(batch):
    chunk = batch // NSUB
    nvec = chunk // LANES
    mesh = plsc.VectorSubcoreMesh(core_axis_name="c", subcore_axis_name="s",
                                  num_cores=1)

    @functools.partial(
        pl.kernel,
        mesh=mesh,
        out_type=jax.ShapeDtypeStruct((LANES,), jnp.float32),
        compiler_params=pltpu.CompilerParams(needs_layout_passes=False),
        scratch_types=[
            pltpu.VMEM((chunk,), jnp.float32),
            pltpu.VMEM((chunk,), jnp.int32),
            pltpu.VMEM((chunk,), jnp.int32),
            pltpu.VMEM((LANES * NBINS,), jnp.float32),
            pltpu.VMEM((LANES * NBINS,), jnp.float32),
            pltpu.VMEM((2 * NBINS,), jnp.float32),
            pltpu.VMEM((NSUB * 2 * NBINS,), jnp.float32),
            pltpu.VMEM((LANES,), jnp.float32),
            pltpu.VMEM_SHARED((NSUB * 2 * NBINS,), jnp.float32),
            pltpu.SemaphoreType.DMA,
            pltpu.SemaphoreType.DMA,
            pltpu.SemaphoreType.DMA,
        ],
    )
    def sc_loss(probs_hbm, labels_hbm, groups_hbm, out_hbm,
                pv, lv, gv, hsa, hca, tsc, rb, ov, sh,
                sem0, sem1, sem2):
        sid = lax.axis_index("s")
        lane = lax.iota(jnp.int32, LANES)
        zeros = jnp.zeros((LANES,), jnp.float32)
        ones = jnp.ones((LANES,), jnp.float32)
        nquart = NBINS // LANES

        base = sid * chunk
        cp0 = pltpu.make_async_copy(probs_hbm.at[pl.ds(base, chunk)], pv, sem0)
        cp1 = pltpu.make_async_copy(labels_hbm.at[pl.ds(base, chunk)], lv, sem1)
        cp2 = pltpu.make_async_copy(groups_hbm.at[pl.ds(base, chunk)], gv, sem2)
        cp0.start()
        cp1.start()
        cp2.start()

        for v in range(LANES * nquart):
            hsa[pl.ds(v * LANES, LANES)] = zeros
            hca[pl.ds(v * LANES, LANES)] = zeros

        cp0.wait()
        cp1.wait()
        cp2.wait()

        lane_off = lane * NBINS

        def step(i, carry):
            for u in range(4):
                off = (i * 4 + u) * LANES
                p = pv[pl.ds(off, LANES)]
                lb = lv[pl.ds(off, LANES)]
                g = gv[pl.ds(off, LANES)]
                slot = lane_off + g + lb * NUM_GROUPS
                plsc.addupdate_scatter(hsa, [slot], p)
                plsc.addupdate_scatter(hca, [slot], ones)
            return carry

        lax.fori_loop(0, nvec // 4, step, 0)

        for j in range(nquart):
            sacc = hsa[pl.ds(j * LANES, LANES)]
            cacc = hca[pl.ds(j * LANES, LANES)]
            for r in range(1, LANES):
                o = r * NBINS + j * LANES
                sacc = sacc + hsa[pl.ds(o, LANES)]
                cacc = cacc + hca[pl.ds(o, LANES)]
            tsc[pl.ds(j * LANES, LANES)] = sacc
            tsc[pl.ds(NBINS + j * LANES, LANES)] = cacc

        pltpu.make_async_copy(
            tsc, sh.at[pl.ds(sid * 2 * NBINS, 2 * NBINS)], sem0).start()
        pltpu.make_async_copy(
            tsc, sh.at[pl.ds(sid * 2 * NBINS, 2 * NBINS)], sem0).wait()

        plsc.subcore_barrier()

        @pl.when(sid == 0)
        def _():
            cbs = pltpu.make_async_copy(sh, rb, sem0)
            cbs.start()
            cbs.wait()
            svecs = []
            cvecs = []
            for j in range(nquart):
                sacc = rb[pl.ds(j * LANES, LANES)]
                cacc = rb[pl.ds(NBINS + j * LANES, LANES)]
                for k in range(1, NSUB):
                    o = k * 2 * NBINS + j * LANES
                    sacc = sacc + rb[pl.ds(o, LANES)]
                    cacc = cacc + rb[pl.ds(o + NBINS, LANES)]
                svecs.append(sacc)
                cvecs.append(cacc)
            neg_var, have_neg = _side_stats(svecs[0], svecs[1],
                                            cvecs[0], cvecs[1])
            pos_var, have_pos = _side_stats(svecs[2], svecs[3],
                                            cvecs[2], cvecs[3])
            zerov = jnp.zeros((LANES,), jnp.float32)
            loss_v = jnp.where(
                have_pos & have_neg,
                jnp.float32(0.5) * (pos_var + neg_var),
                jnp.where(have_pos, pos_var,
                          jnp.where(have_neg, neg_var, zerov)))
            ov[...] = loss_v
            pltpu.sync_copy(ov, out_hbm)

    return sc_loss


def kernel(probs, labels, groups, indices):
    del indices
    out = _make_sc_call(probs.shape[0])(probs, labels, groups)
    return out[0]
